# baseline (speedup 1.0000x reference)
import functools

import jax
import jax.numpy as jnp
from jax import lax
from jax.experimental import pallas as pl
from jax.experimental.pallas import tpu as pltpu
from jax.experimental.pallas import tpu_sc as plsc

BATCH = 1024
SEQ = 200
D = 128
N = BATCH * SEQ

CHUNK = 128
SUB = 2
NBUF = 3
DEPTH = 2


@functools.lru_cache(maxsize=None)
def _build(n_rows, d):
    info = plsc.get_sparse_core_info()
    nc, ns = info.num_cores, info.num_subcores
    nw = nc * ns
    per_w = n_rows // nw
    rows_per_chunk = SUB * CHUNK
    n_chunks = per_w // rows_per_chunk
    n_steady = NBUF * (n_chunks // NBUF)
    assert DEPTH < NBUF

    mesh = plsc.VectorSubcoreMesh(core_axis_name="c", subcore_axis_name="s")

    @functools.partial(
        pl.kernel,
        mesh=mesh,
        out_type=jax.ShapeDtypeStruct((nw, n_chunks, SUB * CHUNK, d),
                                      jnp.float32),
        scratch_types=[
            pltpu.VMEM((per_w,), jnp.int32),
        ] + [pltpu.VMEM((SUB * CHUNK, d), jnp.float32)] * NBUF
          + [pltpu.SemaphoreType.DMA] * (2 * NBUF),
    )
    def gather_kernel(idx_hbm, table_hbm, out_hbm, idx_v, *bufs_and_sems):
        rows = bufs_and_sems[:NBUF]
        gsems = bufs_and_sems[NBUF:2 * NBUF]
        ssems = bufs_and_sems[2 * NBUF:]

        wid = lax.axis_index("s") * nc + lax.axis_index("c")

        rpc = SUB * CHUNK

        def gstart(g, b):
            pltpu.async_copy(
                table_hbm.at[idx_v.at[pl.ds(rpc * g, rpc)]], rows[b],
                gsems[b])

        def gwait(g, b):
            pltpu.make_async_copy(
                table_hbm.at[idx_v.at[pl.ds(rpc * g, rpc)]], rows[b],
                gsems[b]).wait()

        def sstart(g, b):
            pltpu.async_copy(rows[b], out_hbm.at[wid, g], ssems[b])

        def swait(g, b):
            pltpu.make_async_copy(
                rows[b], out_hbm.at[wid, g], ssems[b]).wait()

        pltpu.sync_copy(idx_hbm.at[wid], idx_v)

        for g in range(DEPTH):
            gstart(g, g % NBUF)

        for g in range(NBUF):
            gwait(g, g)
            sstart(g, g)
            gi = g + DEPTH
            if gi < n_chunks:
                if gi - NBUF >= 0:
                    swait(gi - NBUF, gi % NBUF)
                gstart(gi, gi % NBUF)

        def outer(k, carry):
            gg = NBUF * k
            for j in range(NBUF):
                g = gg + j
                b = j
                nb = (j + DEPTH) % NBUF
                gwait(g, b)
                sstart(g, b)
                swait(g + DEPTH - NBUF, nb)

                @pl.when(g + DEPTH < n_chunks)
                def _():
                    gstart(g + DEPTH, nb)
            return carry

        lax.fori_loop(1, n_chunks // NBUF, outer, 0)

        for g in range(n_steady, n_chunks):
            gwait(g, g % NBUF)
            sstart(g, g % NBUF)

        for g in range(n_steady + DEPTH - NBUF, n_chunks):
            swait(g, g % NBUF)

    return gather_kernel


def kernel(enc_input, table):
    nw = 32
    idx = enc_input.reshape(nw, -1).astype(jnp.int32)
    out = _build(N, D)(idx, table)
    return out.reshape(BATCH, SEQ, D)


# speedup vs baseline: 1.0163x; 1.0163x over previous
import functools

import jax
import jax.numpy as jnp
from jax import lax
from jax.experimental import pallas as pl
from jax.experimental.pallas import tpu as pltpu
from jax.experimental.pallas import tpu_sc as plsc

BATCH = 1024
SEQ = 200
D = 128
N = BATCH * SEQ

CHUNK = 128
SUB = 1
NBUF = 6
DEPTH = 4


@functools.lru_cache(maxsize=None)
def _build(n_rows, d):
    info = plsc.get_sparse_core_info()
    nc, ns = info.num_cores, info.num_subcores
    nw = nc * ns
    per_w = n_rows // nw
    rows_per_chunk = SUB * CHUNK
    n_chunks = per_w // rows_per_chunk
    n_steady = NBUF * (n_chunks // NBUF)
    assert DEPTH < NBUF

    mesh = plsc.VectorSubcoreMesh(core_axis_name="c", subcore_axis_name="s")

    @functools.partial(
        pl.kernel,
        mesh=mesh,
        out_type=jax.ShapeDtypeStruct((nw, n_chunks, SUB * CHUNK, d),
                                      jnp.float32),
        scratch_types=[
            pltpu.VMEM((per_w,), jnp.int32),
        ] + [pltpu.VMEM((SUB * CHUNK, d), jnp.float32)] * NBUF
          + [pltpu.SemaphoreType.DMA] * (2 * NBUF),
    )
    def gather_kernel(idx_hbm, table_hbm, out_hbm, idx_v, *bufs_and_sems):
        rows = bufs_and_sems[:NBUF]
        gsems = bufs_and_sems[NBUF:2 * NBUF]
        ssems = bufs_and_sems[2 * NBUF:]

        wid = lax.axis_index("s") * nc + lax.axis_index("c")

        rpc = SUB * CHUNK

        def gstart(g, b):
            pltpu.async_copy(
                table_hbm.at[idx_v.at[pl.ds(rpc * g, rpc)]], rows[b],
                gsems[b])

        def gwait(g, b):
            pltpu.make_async_copy(
                table_hbm.at[idx_v.at[pl.ds(rpc * g, rpc)]], rows[b],
                gsems[b]).wait()

        def sstart(g, b):
            pltpu.async_copy(rows[b], out_hbm.at[wid, g], ssems[b])

        def swait(g, b):
            pltpu.make_async_copy(
                rows[b], out_hbm.at[wid, g], ssems[b]).wait()

        pltpu.sync_copy(idx_hbm.at[wid], idx_v)

        for g in range(DEPTH):
            gstart(g, g % NBUF)

        for g in range(NBUF):
            gwait(g, g)
            sstart(g, g)
            gi = g + DEPTH
            if gi < n_chunks:
                if gi - NBUF >= 0:
                    swait(gi - NBUF, gi % NBUF)
                gstart(gi, gi % NBUF)

        def outer(k, carry):
            gg = NBUF * k
            for j in range(NBUF):
                g = gg + j
                b = j
                nb = (j + DEPTH) % NBUF
                gwait(g, b)
                sstart(g, b)
                swait(g + DEPTH - NBUF, nb)

                @pl.when(g + DEPTH < n_chunks)
                def _():
                    gstart(g + DEPTH, nb)
            return carry

        lax.fori_loop(1, n_chunks // NBUF, outer, 0)

        for g in range(n_steady, n_chunks):
            gwait(g, g % NBUF)
            sstart(g, g % NBUF)

        for g in range(n_steady + DEPTH - NBUF, n_chunks):
            swait(g, g % NBUF)

    return gather_kernel


def kernel(enc_input, table):
    nw = 32
    idx = enc_input.reshape(nw, -1).astype(jnp.int32)
    out = _build(N, D)(idx, table)
    return out.reshape(BATCH, SEQ, D)


# speedup vs baseline: 1.0226x; 1.0062x over previous
import functools

import jax
import jax.numpy as jnp
from jax import lax
from jax.experimental import pallas as pl
from jax.experimental.pallas import tpu as pltpu
from jax.experimental.pallas import tpu_sc as plsc

BATCH = 1024
SEQ = 200
D = 128
N = BATCH * SEQ

CHUNK = 128
SUB = 1
NBUF = 7
DEPTH = 5


@functools.lru_cache(maxsize=None)
def _build(n_rows, d):
    info = plsc.get_sparse_core_info()
    nc, ns = info.num_cores, info.num_subcores
    nw = nc * ns
    per_w = n_rows // nw
    rows_per_chunk = SUB * CHUNK
    n_chunks = per_w // rows_per_chunk
    n_steady = NBUF * (n_chunks // NBUF)
    assert DEPTH < NBUF

    mesh = plsc.VectorSubcoreMesh(core_axis_name="c", subcore_axis_name="s")

    @functools.partial(
        pl.kernel,
        mesh=mesh,
        out_type=jax.ShapeDtypeStruct((nw, n_chunks, SUB * CHUNK, d),
                                      jnp.float32),
        scratch_types=[
            pltpu.VMEM((per_w,), jnp.int32),
        ] + [pltpu.VMEM((SUB * CHUNK, d), jnp.float32)] * NBUF
          + [pltpu.SemaphoreType.DMA] * (2 * NBUF),
    )
    def gather_kernel(idx_hbm, table_hbm, out_hbm, idx_v, *bufs_and_sems):
        rows = bufs_and_sems[:NBUF]
        gsems = bufs_and_sems[NBUF:2 * NBUF]
        ssems = bufs_and_sems[2 * NBUF:]

        wid = lax.axis_index("s") * nc + lax.axis_index("c")

        rpc = SUB * CHUNK

        def gstart(g, b):
            pltpu.async_copy(
                table_hbm.at[idx_v.at[pl.ds(rpc * g, rpc)]], rows[b],
                gsems[b])

        def gwait(g, b):
            pltpu.make_async_copy(
                table_hbm.at[idx_v.at[pl.ds(rpc * g, rpc)]], rows[b],
                gsems[b]).wait()

        def sstart(g, b):
            pltpu.async_copy(rows[b], out_hbm.at[wid, g], ssems[b])

        def swait(g, b):
            pltpu.make_async_copy(
                rows[b], out_hbm.at[wid, g], ssems[b]).wait()

        pltpu.sync_copy(idx_hbm.at[wid], idx_v)

        for g in range(DEPTH):
            gstart(g, g % NBUF)

        for g in range(NBUF):
            gwait(g, g)
            sstart(g, g)
            gi = g + DEPTH
            if gi < n_chunks:
                if gi - NBUF >= 0:
                    swait(gi - NBUF, gi % NBUF)
                gstart(gi, gi % NBUF)

        def outer(k, carry):
            gg = NBUF * k
            for j in range(NBUF):
                g = gg + j
                b = j
                nb = (j + DEPTH) % NBUF
                gwait(g, b)
                sstart(g, b)
                swait(g + DEPTH - NBUF, nb)

                @pl.when(g + DEPTH < n_chunks)
                def _():
                    gstart(g + DEPTH, nb)
            return carry

        lax.fori_loop(1, n_chunks // NBUF, outer, 0)

        for g in range(n_steady, n_chunks):
            gwait(g, g % NBUF)
            sstart(g, g % NBUF)

        for g in range(n_steady + DEPTH - NBUF, n_chunks):
            swait(g, g % NBUF)

    return gather_kernel


def kernel(enc_input, table):
    nw = 32
    idx = enc_input.reshape(nw, -1).astype(jnp.int32)
    out = _build(N, D)(idx, table)
    return out.reshape(BATCH, SEQ, D)


# speedup vs baseline: 1.0272x; 1.0045x over previous
import functools

import jax
import jax.numpy as jnp
from jax import lax
from jax.experimental import pallas as pl
from jax.experimental.pallas import tpu as pltpu
from jax.experimental.pallas import tpu_sc as plsc

BATCH = 1024
SEQ = 200
D = 128
N = BATCH * SEQ

CHUNK = 64
SUB = 1
NBUF = 14
DEPTH = 10


@functools.lru_cache(maxsize=None)
def _build(n_rows, d):
    info = plsc.get_sparse_core_info()
    nc, ns = info.num_cores, info.num_subcores
    nw = nc * ns
    per_w = n_rows // nw
    rows_per_chunk = SUB * CHUNK
    n_chunks = per_w // rows_per_chunk
    n_steady = NBUF * (n_chunks // NBUF)
    assert DEPTH < NBUF

    mesh = plsc.VectorSubcoreMesh(core_axis_name="c", subcore_axis_name="s")

    @functools.partial(
        pl.kernel,
        mesh=mesh,
        out_type=jax.ShapeDtypeStruct((nw, n_chunks, SUB * CHUNK, d),
                                      jnp.float32),
        scratch_types=[
            pltpu.VMEM((per_w,), jnp.int32),
        ] + [pltpu.VMEM((SUB * CHUNK, d), jnp.float32)] * NBUF
          + [pltpu.SemaphoreType.DMA] * (2 * NBUF),
    )
    def gather_kernel(idx_hbm, table_hbm, out_hbm, idx_v, *bufs_and_sems):
        rows = bufs_and_sems[:NBUF]
        gsems = bufs_and_sems[NBUF:2 * NBUF]
        ssems = bufs_and_sems[2 * NBUF:]

        wid = lax.axis_index("s") * nc + lax.axis_index("c")

        rpc = SUB * CHUNK

        def gstart(g, b):
            pltpu.async_copy(
                table_hbm.at[idx_v.at[pl.ds(rpc * g, rpc)]], rows[b],
                gsems[b])

        def gwait(g, b):
            pltpu.make_async_copy(
                table_hbm.at[idx_v.at[pl.ds(rpc * g, rpc)]], rows[b],
                gsems[b]).wait()

        def sstart(g, b):
            pltpu.async_copy(rows[b], out_hbm.at[wid, g], ssems[b])

        def swait(g, b):
            pltpu.make_async_copy(
                rows[b], out_hbm.at[wid, g], ssems[b]).wait()

        pltpu.sync_copy(idx_hbm.at[wid], idx_v)

        for g in range(DEPTH):
            gstart(g, g % NBUF)

        for g in range(NBUF):
            gwait(g, g)
            sstart(g, g)
            gi = g + DEPTH
            if gi < n_chunks:
                if gi - NBUF >= 0:
                    swait(gi - NBUF, gi % NBUF)
                gstart(gi, gi % NBUF)

        def outer(k, carry):
            gg = NBUF * k
            for j in range(NBUF):
                g = gg + j
                b = j
                nb = (j + DEPTH) % NBUF
                gwait(g, b)
                sstart(g, b)
                swait(g + DEPTH - NBUF, nb)

                @pl.when(g + DEPTH < n_chunks)
                def _():
                    gstart(g + DEPTH, nb)
            return carry

        lax.fori_loop(1, n_chunks // NBUF, outer, 0)

        for g in range(n_steady, n_chunks):
            gwait(g, g % NBUF)
            sstart(g, g % NBUF)

        for g in range(n_steady + DEPTH - NBUF, n_chunks):
            swait(g, g % NBUF)

    return gather_kernel


def kernel(enc_input, table):
    nw = 32
    idx = enc_input.reshape(nw, -1).astype(jnp.int32)
    out = _build(N, D)(idx, table)
    return out.reshape(BATCH, SEQ, D)
